# baseline (speedup 1.0000x reference)
import functools

import jax
import jax.numpy as jnp
from jax import lax
from jax.experimental import pallas as pl
from jax.experimental.pallas import tpu as pltpu
from jax.experimental.pallas import tpu_sc as plsc

EMB = 128
HID = 256
NGRP = 256
LAYERS = 3

CS = 3200
G = 128


def _sc_mesh():
    return plsc.VectorSubcoreMesh(core_axis_name="c", subcore_axis_name="s")


def _permute_kernel(e, l_pad):
    assert e % 32 == 0
    per_w = e // 32
    nfull = per_w // G
    tail = per_w - nfull * G

    scr = [pltpu.VMEM((G,), jnp.int32)] * 4
    if tail:
        assert tail % 8 == 0
        scr += [pltpu.VMEM((tail,), jnp.int32)] * 3
        scr += [pltpu.VMEM((max(tail, 16),), jnp.int32)]

    @functools.partial(
        pl.kernel,
        out_type=(jax.ShapeDtypeStruct((l_pad,), jnp.int32),
                  jax.ShapeDtypeStruct((l_pad,), jnp.int32),
                  jax.ShapeDtypeStruct((l_pad,), jnp.int32)),
        mesh=_sc_mesh(),
        scratch_types=scr,
    )
    def k(src_h, rel_h, pos_h, srcg_h, relg_h, eidg_h, *bufs):
        cid = lax.axis_index("c")
        sid = lax.axis_index("s")
        w = cid * 16 + sid
        base_w = w * per_w

        def do_batch(base, n, bufp, bufa, bufb, bufe):
            pltpu.sync_copy(pos_h.at[pl.ds(base, n)], bufp)
            pltpu.sync_copy(src_h.at[pl.ds(base, n)], bufa)
            pltpu.sync_copy(rel_h.at[pl.ds(base, n)], bufb)
            for i in range((n + 15) // 16):
                bufe[pl.ds(i * 16, 16)] = (
                    base + i * 16 + lax.broadcasted_iota(jnp.int32, (16,), 0))
            pltpu.sync_copy(bufa, srcg_h.at[bufp])
            pltpu.sync_copy(bufb, relg_h.at[bufp])
            be = bufe if n == bufe.shape[0] else bufe.at[pl.ds(0, n)]
            pltpu.sync_copy(be, eidg_h.at[bufp])

        def body(j, carry):
            do_batch(base_w + j * G, G, *bufs[:4])
            return carry

        lax.fori_loop(0, nfull, body, 0)
        if tail:
            do_batch(base_w + nfull * G, tail, *bufs[4:8])

    return k


_D = 4
_PIPE = True


def _segsum_ba_kernel(t_rows, l_pad, n_out, init_small):
    nc = n_out // CS
    assert n_out % CS == 0 and nc % 2 == 0
    rows_t = CS // 16

    scr = ([pltpu.VMEM((128,), jnp.int32)]
           + [pltpu.VMEM((G,), jnp.int32) for _ in range(2 * _D)]
           + [pltpu.VMEM((G, EMB), jnp.float32) for _ in range(_D)]
           + [pltpu.VMEM_SHARED((CS + 1, EMB), jnp.float32)]
           + [pltpu.SemaphoreType.DMA] * (3 * _D))

    @functools.partial(
        pl.kernel,
        out_type=jax.ShapeDtypeStruct((n_out, EMB), jnp.float32),
        mesh=_sc_mesh(),
        scratch_types=scr,
    )
    def k(table_h, idxg_h, relg_h, tab_h, init_h, out_h, tab_v, *rest):
        idx_v = rest[0:_D]
        rel_v = rest[_D:2 * _D]
        rows_v = rest[2 * _D:3 * _D]
        acc = rest[3 * _D]
        sem_i = rest[3 * _D + 1:3 * _D + 1 + _D]
        sem_g = rest[3 * _D + 1 + _D:3 * _D + 1 + 2 * _D]
        sem_s = rest[3 * _D + 1 + 2 * _D:3 * _D + 1 + 3 * _D]
        cid = lax.axis_index("c")
        sid = lax.axis_index("s")
        pltpu.sync_copy(tab_h, tab_v)

        for cl in range(nc // 2):
            c = cl * 2 + cid
            g0 = (2 * cl) // 16
            l0 = (2 * cl) % 16
            sg = tab_v[pl.ds(g0 * 16, 16)]
            cg = tab_v[pl.ds(64 + g0 * 16, 16)]
            start = pl.multiple_of(jnp.where(cid == 1, sg[l0 + 1], sg[l0]), G)
            cnt = jnp.where(cid == 1, cg[l0 + 1], cg[l0])
            if init_small:
                pltpu.sync_copy(init_h,
                                acc.at[pl.ds(sid * rows_t, rows_t)])
            else:
                pltpu.sync_copy(
                    init_h.at[pl.ds(c * CS + sid * rows_t, rows_t)],
                    acc.at[pl.ds(sid * rows_t, rows_t)])
            plsc.subcore_barrier()

            nb = (cnt + (G - 1)) // G
            nb_t = (jnp.maximum(nb - sid, 0) + 15) // 16

            def emit(i, r):
                @pl.when(i < nb_t)
                def _():
                    @pl.when(i >= _D)
                    def _():
                        pltpu.make_async_copy(
                            rows_v[r], acc.at[rel_v[r]], sem_s[r]).wait()
                    j = sid + 16 * i
                    base = start + j * G
                    pltpu.async_copy(idxg_h.at[pl.ds(base, G)],
                                     idx_v[r], sem_i[r])
                    pltpu.async_copy(relg_h.at[pl.ds(base, G)],
                                     rel_v[r], sem_i[r])
                r1 = (r - 1) % _D
                @pl.when((i >= 1) & (i <= nb_t))
                def _():
                    m = i - 1
                    pltpu.make_async_copy(idxg_h.at[pl.ds(0, G)],
                                          idx_v[r1], sem_i[r1]).wait()
                    pltpu.make_async_copy(relg_h.at[pl.ds(0, G)],
                                          rel_v[r1], sem_i[r1]).wait()
                    j = sid + 16 * m
                    for q in range(G // 16):
                        off = (j * G + q * 16
                               + lax.broadcasted_iota(jnp.int32, (16,), 0))
                        valid = off < cnt
                        sl = pl.ds(q * 16, 16)
                        idx_v[r1][sl] = jnp.where(valid, idx_v[r1][sl], 0)
                        rel_v[r1][sl] = jnp.where(valid, rel_v[r1][sl], CS)
                    pltpu.async_copy(table_h.at[idx_v[r1]], rows_v[r1],
                                     sem_g[r1])
                r2 = (r - 2) % _D
                @pl.when((i >= 2) & (i <= nb_t + 1))
                def _():
                    pltpu.make_async_copy(table_h.at[idx_v[r2]],
                                          rows_v[r2], sem_g[r2]).wait()
                    pltpu.async_copy(rows_v[r2], acc.at[rel_v[r2]],
                                     sem_s[r2], add=True)

            def body(gi, carry):
                for r in range(_D):
                    emit(gi * _D + r, r)
                return carry

            if _PIPE:
                ngroups = (nb_t + 2 + (_D - 1)) // _D
                lax.fori_loop(0, ngroups, body, 0)
            else:
                def sbody(m, carry):
                    j = sid + 16 * m
                    base = start + j * G
                    pltpu.sync_copy(idxg_h.at[pl.ds(base, G)], idx_v[0])
                    pltpu.sync_copy(relg_h.at[pl.ds(base, G)], rel_v[0])
                    for q in range(G // 16):
                        off = (j * G + q * 16
                               + lax.broadcasted_iota(jnp.int32, (16,), 0))
                        valid = off < cnt
                        sl = pl.ds(q * 16, 16)
                        idx_v[0][sl] = jnp.where(valid, idx_v[0][sl], 0)
                        rel_v[0][sl] = jnp.where(valid, rel_v[0][sl], CS)
                    pltpu.async_copy(table_h.at[idx_v[0]], rows_v[0],
                                     sem_g[0]).wait()
                    pltpu.sync_copy(rows_v[0], acc.at[rel_v[0]], add=True)
                    return carry
                lax.fori_loop(0, nb_t, sbody, 0)
            if _PIPE:
                for s in range(_D):
                    @pl.when(nb_t > s)
                    def _():
                        pltpu.make_async_copy(
                            rows_v[s], acc.at[rel_v[s]], sem_s[s]).wait()
            plsc.subcore_barrier()
            pltpu.sync_copy(acc.at[pl.ds(sid * rows_t, rows_t)],
                            out_h.at[pl.ds(c * CS + sid * rows_t, rows_t)])
            plsc.subcore_barrier()

    return k


def _counts_body(batch_ref, row_ref, col_ref):
    i = pl.program_id(0)

    @pl.when(i == 0)
    def _():
        row_ref[...] = jnp.zeros_like(row_ref)
        col_ref[...] = jnp.zeros_like(col_ref)

    b = batch_ref[...]
    onehot = (b == lax.broadcasted_iota(jnp.int32, (1, NGRP), 1)).astype(jnp.float32)
    row_ref[0:1, :] += jnp.sum(onehot, axis=0, keepdims=True)
    col_ref[...] += lax.dot_general(
        onehot, jnp.ones((onehot.shape[0], 128), jnp.float32),
        (((0,), (0,)), ((), ())), preferred_element_type=jnp.float32)


def _counts(batch2d):
    n = batch2d.shape[0]
    B = 2048
    npad = pl.cdiv(n, B) * B
    bpad = jnp.full((npad, 1), NGRP, jnp.int32).at[:n].set(batch2d)
    return pl.pallas_call(
        _counts_body,
        grid=(npad // B,),
        in_specs=[pl.BlockSpec((B, 1), lambda i: (i, 0))],
        out_specs=[pl.BlockSpec((8, NGRP), lambda i: (0, 0)),
                   pl.BlockSpec((NGRP, 128), lambda i: (0, 0))],
        out_shape=[jax.ShapeDtypeStruct((8, NGRP), jnp.float32),
                   jax.ShapeDtypeStruct((NGRP, 128), jnp.float32)],
    )(bpad)


def _scale_body(batch_ref, cmat_ref, o_ref):
    b = batch_ref[...]
    onehot = (b == lax.broadcasted_iota(jnp.int32, (1, NGRP), 1)).astype(jnp.float32)
    rs = lax.rsqrt(jnp.maximum(cmat_ref[...][:, 0:1], 1.0))
    o_ref[...] = lax.dot_general(onehot, rs, (((1,), (0,)), ((), ())),
                                 preferred_element_type=jnp.float32)


def _node_scale(batch2d, cmat):
    n = batch2d.shape[0]
    B = 2048
    npad = pl.cdiv(n, B) * B
    bpad = jnp.full((npad, 1), NGRP, jnp.int32).at[:n].set(batch2d)
    out = pl.pallas_call(
        _scale_body,
        grid=(npad // B,),
        in_specs=[pl.BlockSpec((B, 1), lambda i: (i, 0)),
                  pl.BlockSpec((NGRP, 128), lambda i: (0, 0))],
        out_specs=pl.BlockSpec((B, 1), lambda i: (i, 0)),
        out_shape=jax.ShapeDtypeStruct((npad, 1), jnp.float32),
    )(bpad, cmat)
    return out[:n]


def _mlp_body(two_agg, agg_ref, agg2_ref, x_ref, scale_ref, W1_ref, b1_ref,
              W2_ref, b2_ref, g_ref, be_ref, o_ref, *, last_act):
    agg = agg_ref[...]
    if two_agg:
        agg = agg + agg2_ref[...]
    u = lax.dot_general(agg, W1_ref[...], (((1,), (0,)), ((), ())),
                        preferred_element_type=jnp.float32) + b1_ref[...]
    u = jnp.maximum(u, 0.0)
    h = lax.dot_general(u, W2_ref[...], (((1,), (0,)), ((), ())),
                        preferred_element_type=jnp.float32) + b2_ref[...]
    mu = jnp.mean(h, axis=1, keepdims=True)
    var = jnp.mean((h - mu) * (h - mu), axis=1, keepdims=True)
    h = (h - mu) * lax.rsqrt(var + 1e-5) * g_ref[...] + be_ref[...]
    h = h * scale_ref[...]
    if last_act:
        h = jnp.maximum(h, 0.0)
    o_ref[...] = h + x_ref[...]


def _mlp(aggs, x, scale, W1, b1, W2, b2, g, be, last_act):
    n = x.shape[0]
    B = 1024
    two = len(aggs) == 2
    body = functools.partial(_mlp_body, two, last_act=last_act)
    if not two:
        def body(agg_ref, *rest, _b=functools.partial(_mlp_body, False,
                                                      last_act=last_act)):
            _b(agg_ref, agg_ref, *rest)
    row = pl.BlockSpec((B, EMB), lambda i: (i, 0))
    return pl.pallas_call(
        body,
        grid=(pl.cdiv(n, B),),
        in_specs=([row] * (2 if two else 1)
                  + [row,
                     pl.BlockSpec((B, 1), lambda i: (i, 0)),
                     pl.BlockSpec((EMB, HID), lambda i: (0, 0)),
                     pl.BlockSpec((1, HID), lambda i: (0, 0)),
                     pl.BlockSpec((HID, EMB), lambda i: (0, 0)),
                     pl.BlockSpec((1, EMB), lambda i: (0, 0)),
                     pl.BlockSpec((1, EMB), lambda i: (0, 0)),
                     pl.BlockSpec((1, EMB), lambda i: (0, 0))]),
        out_specs=row,
        out_shape=jax.ShapeDtypeStruct((n, EMB), jnp.float32),
    )(*aggs, x, scale, W1, b1.reshape(1, HID), W2, b2.reshape(1, EMB),
      g.reshape(1, EMB), be.reshape(1, EMB))


def _pool_body(x_ref, batch_ref, cmat_ref, o_ref, *, nrows, nblocks, B):
    i = pl.program_id(0)

    @pl.when(i == 0)
    def _():
        o_ref[...] = jnp.zeros_like(o_ref)

    rowid = i * B + lax.broadcasted_iota(jnp.int32, (B, 1), 0)
    xm = jnp.where(rowid < nrows, x_ref[...], 0.0)
    b = batch_ref[...]
    onehot = (b == lax.broadcasted_iota(jnp.int32, (1, NGRP), 1)).astype(jnp.float32)
    o_ref[...] += lax.dot_general(onehot, xm, (((0,), (0,)), ((), ())),
                                  preferred_element_type=jnp.float32)

    @pl.when(i == nblocks - 1)
    def _():
        o_ref[...] = o_ref[...] / jnp.maximum(cmat_ref[...], 1.0)


def _pool(x, batch2d, cmat):
    n = x.shape[0]
    B = 2048
    npad = pl.cdiv(n, B) * B
    nblocks = npad // B
    bpad = jnp.full((npad, 1), NGRP, jnp.int32).at[:n].set(batch2d)
    body = functools.partial(_pool_body, nrows=n, nblocks=nblocks, B=B)
    return pl.pallas_call(
        body,
        grid=(nblocks,),
        in_specs=[pl.BlockSpec((B, EMB), lambda i: (i, 0)),
                  pl.BlockSpec((B, 1), lambda i: (i, 0)),
                  pl.BlockSpec((NGRP, 128), lambda i: (0, 0))],
        out_specs=pl.BlockSpec((NGRP, EMB), lambda i: (0, 0)),
        out_shape=jax.ShapeDtypeStruct((NGRP, EMB), jnp.float32),
    )(x, bpad, cmat)


def _bucket(dst, e, n_eff):
    nc = n_eff // CS
    assert nc <= 64
    c_e = dst // CS
    oh = (c_e[:, None] == jnp.arange(nc, dtype=jnp.int32)[None, :]).astype(jnp.int32)
    rank = jnp.sum((jnp.cumsum(oh, axis=0) - oh) * oh, axis=1)
    cnt = jnp.sum(oh, axis=0).astype(jnp.int32)
    cnt_pad = ((cnt + (G - 1)) // G) * G
    starts = jnp.concatenate([jnp.zeros((1,), jnp.int32),
                              jnp.cumsum(cnt_pad)[:-1].astype(jnp.int32)])
    pos = (jnp.sum(starts[None, :] * oh, axis=1) + rank).astype(jnp.int32)
    rel = (dst - c_e * CS).astype(jnp.int32)
    tab = jnp.zeros((128,), jnp.int32).at[0:nc].set(starts).at[64:64 + nc].set(cnt)
    return tab, rel, pos, e + nc * G


def kernel(ab_x, ab_edge_index, ab_batch, ba_x, ba_edge_index, ba_edge_attr,
           ba_batch, W1, b1, W2, b2, gamma, beta):
    n_ab = ab_x.shape[0]
    n_ba = ba_x.shape[0]
    e_ab = ab_edge_index.shape[1]
    e_ba = ba_edge_index.shape[1]
    ab_src, ab_dst = ab_edge_index[0], ab_edge_index[1]
    ba_src, ba_dst = ba_edge_index[0], ba_edge_index[1]
    ab_batch2 = ab_batch.reshape(-1, 1)
    ba_batch2 = ba_batch.reshape(-1, 1)
    n_ab_eff = ((n_ab + CS - 1) // CS) * CS
    if (n_ab_eff // CS) % 2:
        n_ab_eff += CS

    tab_ba, rel_ba, pos_ba, lpad_ba = _bucket(ba_dst, e_ba, n_ba)
    tab_ab, rel_ab, pos_ab, lpad_ab = _bucket(ab_dst, e_ab, n_ab_eff)

    srcg_ba, relg_ba, eidg_ba = _permute_kernel(e_ba, lpad_ba)(
        ba_src, rel_ba, pos_ba)
    srcg_ab, relg_ab, eidg_ab = _permute_kernel(e_ab, lpad_ab)(
        ab_src, rel_ab, pos_ab)

    zsmall = jnp.zeros((CS // 16, EMB), jnp.float32)
    s_attr = _segsum_ba_kernel(e_ba, lpad_ba, n_ba, True)(
        ba_edge_attr, eidg_ba, relg_ba, tab_ba, zsmall)

    _, cmat_ab = _counts(ab_batch2)
    _, cmat_ba = _counts(ba_batch2)
    scale_ab = _node_scale(ab_batch2, cmat_ab)
    scale_ba = _node_scale(ba_batch2, cmat_ba)

    ba_kern = _segsum_ba_kernel(n_ba, lpad_ba, n_ba, False)
    ab_kern1 = _segsum_ba_kernel(n_ab, lpad_ab, n_ab_eff, True)
    ab_kern2 = _segsum_ba_kernel(n_ba, lpad_ab, n_ab_eff, False)

    node_h, edge_h = ab_x, ba_x
    for l in range(LAYERS):
        last_act = (l != LAYERS - 1)
        part = ab_kern1(node_h, srcg_ab, relg_ab, tab_ab, zsmall)
        agg_ab = ab_kern2(edge_h, eidg_ab, relg_ab, tab_ab, part)
        agg_ba = ba_kern(edge_h, srcg_ba, relg_ba, tab_ba, s_attr)
        node_h = _mlp((agg_ab,), node_h, scale_ab, W1[l], b1[l],
                      W2[l], b2[l], gamma[l], beta[l], last_act)
        edge_h = _mlp((agg_ba,), edge_h, scale_ba, W1[l], b1[l], W2[l],
                      b2[l], gamma[l], beta[l], last_act)

    ab_repr = _pool(node_h, ab_batch2, cmat_ab)
    ba_repr = _pool(edge_h, ba_batch2, cmat_ba)
    return (ab_repr, ba_repr, node_h, edge_h)


# speedup vs baseline: 1.0784x; 1.0784x over previous
import functools

import jax
import jax.numpy as jnp
from jax import lax
from jax.experimental import pallas as pl
from jax.experimental.pallas import tpu as pltpu
from jax.experimental.pallas import tpu_sc as plsc

EMB = 128
HID = 256
NGRP = 256
LAYERS = 3

CS = 6400
G = 128


def _sc_mesh():
    return plsc.VectorSubcoreMesh(core_axis_name="c", subcore_axis_name="s")


def _permute_kernel(e, l_pad):
    assert e % 32 == 0
    per_w = e // 32
    nfull = per_w // G
    tail = per_w - nfull * G
    D = 3

    scr = [pltpu.VMEM((G,), jnp.int32) for _ in range(4 * D)]
    scr += [pltpu.SemaphoreType.DMA] * (2 * D)
    if tail:
        assert tail % 8 == 0
        scr += [pltpu.VMEM((tail,), jnp.int32)] * 3
        scr += [pltpu.VMEM((max(tail, 16),), jnp.int32)]

    @functools.partial(
        pl.kernel,
        out_type=(jax.ShapeDtypeStruct((l_pad,), jnp.int32),
                  jax.ShapeDtypeStruct((l_pad,), jnp.int32),
                  jax.ShapeDtypeStruct((l_pad,), jnp.int32)),
        mesh=_sc_mesh(),
        scratch_types=scr,
    )
    def k(src_h, rel_h, pos_h, srcg_h, relg_h, eidg_h, *bufs):
        bufp = bufs[0:D]
        bufa = bufs[D:2 * D]
        bufb = bufs[2 * D:3 * D]
        bufe = bufs[3 * D:4 * D]
        sem_p = bufs[4 * D:5 * D]
        sem_o = bufs[5 * D:6 * D]
        tailbufs = bufs[6 * D:]
        cid = lax.axis_index("c")
        sid = lax.axis_index("s")
        base_w = (cid * 16 + sid) * per_w

        def emit(i, r):
            @pl.when(i < nfull)
            def _():
                @pl.when(i >= D)
                def _():
                    pltpu.make_async_copy(bufa[r], srcg_h.at[bufp[r]],
                                          sem_o[r]).wait()
                    pltpu.make_async_copy(bufb[r], relg_h.at[bufp[r]],
                                          sem_o[r]).wait()
                    pltpu.make_async_copy(bufe[r], eidg_h.at[bufp[r]],
                                          sem_o[r]).wait()
                base = base_w + i * G
                pltpu.async_copy(pos_h.at[pl.ds(base, G)], bufp[r], sem_p[r])
                pltpu.async_copy(src_h.at[pl.ds(base, G)], bufa[r], sem_p[r])
                pltpu.async_copy(rel_h.at[pl.ds(base, G)], bufb[r], sem_p[r])
                for q in range(G // 16):
                    bufe[r][pl.ds(q * 16, 16)] = (
                        base + q * 16
                        + lax.broadcasted_iota(jnp.int32, (16,), 0))
            r1 = (r - 1) % D
            @pl.when((i >= 1) & (i <= nfull))
            def _():
                for _w in range(3):
                    pltpu.make_async_copy(pos_h.at[pl.ds(0, G)],
                                          bufp[r1], sem_p[r1]).wait()
                pltpu.async_copy(bufa[r1], srcg_h.at[bufp[r1]], sem_o[r1])
                pltpu.async_copy(bufb[r1], relg_h.at[bufp[r1]], sem_o[r1])
                pltpu.async_copy(bufe[r1], eidg_h.at[bufp[r1]], sem_o[r1])

        def body(gi, carry):
            for r in range(D):
                emit(gi * D + r, r)
            return carry

        ngroups = (nfull + 1 + (D - 1)) // D
        lax.fori_loop(0, ngroups, body, 0)
        for sl in range(min(D, nfull)):
            pltpu.make_async_copy(bufa[sl], srcg_h.at[bufp[sl]],
                                  sem_o[sl]).wait()
            pltpu.make_async_copy(bufb[sl], relg_h.at[bufp[sl]],
                                  sem_o[sl]).wait()
            pltpu.make_async_copy(bufe[sl], eidg_h.at[bufp[sl]],
                                  sem_o[sl]).wait()

        if tail:
            base = base_w + nfull * G
            tp, ta, tb, te = tailbufs
            pltpu.sync_copy(pos_h.at[pl.ds(base, tail)], tp)
            pltpu.sync_copy(src_h.at[pl.ds(base, tail)], ta)
            pltpu.sync_copy(rel_h.at[pl.ds(base, tail)], tb)
            for q in range((tail + 15) // 16):
                te[pl.ds(q * 16, 16)] = (
                    base + q * 16 + lax.broadcasted_iota(jnp.int32, (16,), 0))
            pltpu.sync_copy(ta, srcg_h.at[tp])
            pltpu.sync_copy(tb, relg_h.at[tp])
            tes = te if tail == te.shape[0] else te.at[pl.ds(0, tail)]
            pltpu.sync_copy(tes, eidg_h.at[tp])

    return k


_D = 4
_PIPE = True


def _segsum_ba_kernel(t_rows, l_pad, n_out, init_small):
    nc = n_out // CS
    assert n_out % CS == 0 and nc <= 64
    rows_t = CS // 16

    scr = ([pltpu.VMEM((128,), jnp.int32)]
           + [pltpu.VMEM((G,), jnp.int32) for _ in range(2 * _D)]
           + [pltpu.VMEM((G, EMB), jnp.float32) for _ in range(_D)]
           + [pltpu.VMEM_SHARED((CS + 1, EMB), jnp.float32)]
           + [pltpu.SemaphoreType.DMA] * (3 * _D))

    @functools.partial(
        pl.kernel,
        out_type=jax.ShapeDtypeStruct((n_out, EMB), jnp.float32),
        mesh=_sc_mesh(),
        scratch_types=scr,
    )
    def k(table_h, idxg_h, relg_h, tab_h, init_h, out_h, tab_v, *rest):
        idx_v = rest[0:_D]
        rel_v = rest[_D:2 * _D]
        rows_v = rest[2 * _D:3 * _D]
        acc = rest[3 * _D]
        sem_i = rest[3 * _D + 1:3 * _D + 1 + _D]
        sem_g = rest[3 * _D + 1 + _D:3 * _D + 1 + 2 * _D]
        sem_s = rest[3 * _D + 1 + 2 * _D:3 * _D + 1 + 3 * _D]
        cid = lax.axis_index("c")
        sid = lax.axis_index("s")
        pltpu.sync_copy(tab_h, tab_v)

        for cl in range((nc + 1) // 2):
            c = cl * 2 + cid
            g0 = (2 * cl) // 16
            l0 = (2 * cl) % 16
            sg = tab_v[pl.ds(g0 * 16, 16)]
            cg = tab_v[pl.ds(64 + g0 * 16, 16)]
            start = pl.multiple_of(jnp.where(cid == 1, sg[l0 + 1], sg[l0]), G)
            cnt = jnp.where(cid == 1, cg[l0 + 1], cg[l0])
            live = c < nc

            @pl.when(live)
            def _():
                if init_small:
                    pltpu.sync_copy(init_h,
                                    acc.at[pl.ds(sid * rows_t, rows_t)])
                else:
                    pltpu.sync_copy(
                        init_h.at[pl.ds(c * CS + sid * rows_t, rows_t)],
                        acc.at[pl.ds(sid * rows_t, rows_t)])
            plsc.subcore_barrier()

            nb = (cnt + (G - 1)) // G
            nb_t = (jnp.maximum(nb - sid, 0) + 15) // 16

            def emit(i, r):
                @pl.when(i < nb_t)
                def _():
                    @pl.when(i >= _D)
                    def _():
                        pltpu.make_async_copy(
                            rows_v[r], acc.at[rel_v[r]], sem_s[r]).wait()
                    j = sid + 16 * i
                    base = start + j * G
                    pltpu.async_copy(idxg_h.at[pl.ds(base, G)],
                                     idx_v[r], sem_i[r])
                    pltpu.async_copy(relg_h.at[pl.ds(base, G)],
                                     rel_v[r], sem_i[r])
                r1 = (r - 1) % _D
                @pl.when((i >= 1) & (i <= nb_t))
                def _():
                    m = i - 1
                    pltpu.make_async_copy(idxg_h.at[pl.ds(0, G)],
                                          idx_v[r1], sem_i[r1]).wait()
                    pltpu.make_async_copy(relg_h.at[pl.ds(0, G)],
                                          rel_v[r1], sem_i[r1]).wait()
                    j = sid + 16 * m
                    for q in range(G // 16):
                        off = (j * G + q * 16
                               + lax.broadcasted_iota(jnp.int32, (16,), 0))
                        valid = off < cnt
                        sl = pl.ds(q * 16, 16)
                        idx_v[r1][sl] = jnp.where(valid, idx_v[r1][sl], 0)
                        rel_v[r1][sl] = jnp.where(valid, rel_v[r1][sl], CS)
                    pltpu.async_copy(table_h.at[idx_v[r1]], rows_v[r1],
                                     sem_g[r1])
                r2 = (r - 2) % _D
                @pl.when((i >= 2) & (i <= nb_t + 1))
                def _():
                    pltpu.make_async_copy(table_h.at[idx_v[r2]],
                                          rows_v[r2], sem_g[r2]).wait()
                    pltpu.async_copy(rows_v[r2], acc.at[rel_v[r2]],
                                     sem_s[r2], add=True)

            def body(gi, carry):
                for r in range(_D):
                    emit(gi * _D + r, r)
                return carry

            if _PIPE:
                ngroups = (nb_t + 2 + (_D - 1)) // _D
                lax.fori_loop(0, ngroups, body, 0)
            else:
                def sbody(m, carry):
                    j = sid + 16 * m
                    base = start + j * G
                    pltpu.sync_copy(idxg_h.at[pl.ds(base, G)], idx_v[0])
                    pltpu.sync_copy(relg_h.at[pl.ds(base, G)], rel_v[0])
                    for q in range(G // 16):
                        off = (j * G + q * 16
                               + lax.broadcasted_iota(jnp.int32, (16,), 0))
                        valid = off < cnt
                        sl = pl.ds(q * 16, 16)
                        idx_v[0][sl] = jnp.where(valid, idx_v[0][sl], 0)
                        rel_v[0][sl] = jnp.where(valid, rel_v[0][sl], CS)
                    pltpu.async_copy(table_h.at[idx_v[0]], rows_v[0],
                                     sem_g[0]).wait()
                    pltpu.sync_copy(rows_v[0], acc.at[rel_v[0]], add=True)
                    return carry
                lax.fori_loop(0, nb_t, sbody, 0)
            if _PIPE:
                for s in range(_D):
                    @pl.when(nb_t > s)
                    def _():
                        pltpu.make_async_copy(
                            rows_v[s], acc.at[rel_v[s]], sem_s[s]).wait()
            plsc.subcore_barrier()

            @pl.when(live)
            def _():
                pltpu.sync_copy(
                    acc.at[pl.ds(sid * rows_t, rows_t)],
                    out_h.at[pl.ds(c * CS + sid * rows_t, rows_t)])
            plsc.subcore_barrier()

    return k


def _counts_body(batch_ref, row_ref, col_ref):
    i = pl.program_id(0)

    @pl.when(i == 0)
    def _():
        row_ref[...] = jnp.zeros_like(row_ref)
        col_ref[...] = jnp.zeros_like(col_ref)

    b = batch_ref[...]
    onehot = (b == lax.broadcasted_iota(jnp.int32, (1, NGRP), 1)).astype(jnp.float32)
    row_ref[0:1, :] += jnp.sum(onehot, axis=0, keepdims=True)
    col_ref[...] += lax.dot_general(
        onehot, jnp.ones((onehot.shape[0], 128), jnp.float32),
        (((0,), (0,)), ((), ())), preferred_element_type=jnp.float32)


def _counts(batch2d):
    n = batch2d.shape[0]
    B = 2048
    npad = pl.cdiv(n, B) * B
    bpad = jnp.full((npad, 1), NGRP, jnp.int32).at[:n].set(batch2d)
    return pl.pallas_call(
        _counts_body,
        grid=(npad // B,),
        in_specs=[pl.BlockSpec((B, 1), lambda i: (i, 0))],
        out_specs=[pl.BlockSpec((8, NGRP), lambda i: (0, 0)),
                   pl.BlockSpec((NGRP, 128), lambda i: (0, 0))],
        out_shape=[jax.ShapeDtypeStruct((8, NGRP), jnp.float32),
                   jax.ShapeDtypeStruct((NGRP, 128), jnp.float32)],
    )(bpad)


def _scale_body(batch_ref, cmat_ref, o_ref):
    b = batch_ref[...]
    onehot = (b == lax.broadcasted_iota(jnp.int32, (1, NGRP), 1)).astype(jnp.float32)
    rs = lax.rsqrt(jnp.maximum(cmat_ref[...][:, 0:1], 1.0))
    o_ref[...] = lax.dot_general(onehot, rs, (((1,), (0,)), ((), ())),
                                 preferred_element_type=jnp.float32)


def _node_scale(batch2d, cmat):
    n = batch2d.shape[0]
    B = 2048
    npad = pl.cdiv(n, B) * B
    bpad = jnp.full((npad, 1), NGRP, jnp.int32).at[:n].set(batch2d)
    out = pl.pallas_call(
        _scale_body,
        grid=(npad // B,),
        in_specs=[pl.BlockSpec((B, 1), lambda i: (i, 0)),
                  pl.BlockSpec((NGRP, 128), lambda i: (0, 0))],
        out_specs=pl.BlockSpec((B, 1), lambda i: (i, 0)),
        out_shape=jax.ShapeDtypeStruct((npad, 1), jnp.float32),
    )(bpad, cmat)
    return out[:n]


def _mlp_body(two_agg, agg_ref, agg2_ref, x_ref, scale_ref, W1_ref, b1_ref,
              W2_ref, b2_ref, g_ref, be_ref, o_ref, *, last_act):
    agg = agg_ref[...]
    if two_agg:
        agg = agg + agg2_ref[...]
    u = lax.dot_general(agg, W1_ref[...], (((1,), (0,)), ((), ())),
                        preferred_element_type=jnp.float32) + b1_ref[...]
    u = jnp.maximum(u, 0.0)
    h = lax.dot_general(u, W2_ref[...], (((1,), (0,)), ((), ())),
                        preferred_element_type=jnp.float32) + b2_ref[...]
    mu = jnp.mean(h, axis=1, keepdims=True)
    var = jnp.mean((h - mu) * (h - mu), axis=1, keepdims=True)
    h = (h - mu) * lax.rsqrt(var + 1e-5) * g_ref[...] + be_ref[...]
    h = h * scale_ref[...]
    if last_act:
        h = jnp.maximum(h, 0.0)
    o_ref[...] = h + x_ref[...]


def _mlp(aggs, x, scale, W1, b1, W2, b2, g, be, last_act):
    n = x.shape[0]
    B = 1024
    two = len(aggs) == 2
    body = functools.partial(_mlp_body, two, last_act=last_act)
    if not two:
        def body(agg_ref, *rest, _b=functools.partial(_mlp_body, False,
                                                      last_act=last_act)):
            _b(agg_ref, agg_ref, *rest)
    row = pl.BlockSpec((B, EMB), lambda i: (i, 0))
    return pl.pallas_call(
        body,
        grid=(pl.cdiv(n, B),),
        in_specs=([row] * (2 if two else 1)
                  + [row,
                     pl.BlockSpec((B, 1), lambda i: (i, 0)),
                     pl.BlockSpec((EMB, HID), lambda i: (0, 0)),
                     pl.BlockSpec((1, HID), lambda i: (0, 0)),
                     pl.BlockSpec((HID, EMB), lambda i: (0, 0)),
                     pl.BlockSpec((1, EMB), lambda i: (0, 0)),
                     pl.BlockSpec((1, EMB), lambda i: (0, 0)),
                     pl.BlockSpec((1, EMB), lambda i: (0, 0))]),
        out_specs=row,
        out_shape=jax.ShapeDtypeStruct((n, EMB), jnp.float32),
    )(*aggs, x, scale, W1, b1.reshape(1, HID), W2, b2.reshape(1, EMB),
      g.reshape(1, EMB), be.reshape(1, EMB))


def _pool_body(x_ref, batch_ref, cmat_ref, o_ref, *, nrows, nblocks, B):
    i = pl.program_id(0)

    @pl.when(i == 0)
    def _():
        o_ref[...] = jnp.zeros_like(o_ref)

    rowid = i * B + lax.broadcasted_iota(jnp.int32, (B, 1), 0)
    xm = jnp.where(rowid < nrows, x_ref[...], 0.0)
    b = batch_ref[...]
    onehot = (b == lax.broadcasted_iota(jnp.int32, (1, NGRP), 1)).astype(jnp.float32)
    o_ref[...] += lax.dot_general(onehot, xm, (((0,), (0,)), ((), ())),
                                  preferred_element_type=jnp.float32)

    @pl.when(i == nblocks - 1)
    def _():
        o_ref[...] = o_ref[...] / jnp.maximum(cmat_ref[...], 1.0)


def _pool(x, batch2d, cmat):
    n = x.shape[0]
    B = 2048
    npad = pl.cdiv(n, B) * B
    nblocks = npad // B
    bpad = jnp.full((npad, 1), NGRP, jnp.int32).at[:n].set(batch2d)
    body = functools.partial(_pool_body, nrows=n, nblocks=nblocks, B=B)
    return pl.pallas_call(
        body,
        grid=(nblocks,),
        in_specs=[pl.BlockSpec((B, EMB), lambda i: (i, 0)),
                  pl.BlockSpec((B, 1), lambda i: (i, 0)),
                  pl.BlockSpec((NGRP, 128), lambda i: (0, 0))],
        out_specs=pl.BlockSpec((NGRP, EMB), lambda i: (0, 0)),
        out_shape=jax.ShapeDtypeStruct((NGRP, EMB), jnp.float32),
    )(x, bpad, cmat)


def _bucket(dst, e, n_eff):
    nc = n_eff // CS
    assert nc <= 64
    c_e = dst // CS
    oh = (c_e[:, None] == jnp.arange(nc, dtype=jnp.int32)[None, :]).astype(jnp.int32)
    rank = jnp.sum((jnp.cumsum(oh, axis=0) - oh) * oh, axis=1)
    cnt = jnp.sum(oh, axis=0).astype(jnp.int32)
    cnt_pad = ((cnt + (G - 1)) // G) * G
    starts = jnp.concatenate([jnp.zeros((1,), jnp.int32),
                              jnp.cumsum(cnt_pad)[:-1].astype(jnp.int32)])
    pos = (jnp.sum(starts[None, :] * oh, axis=1) + rank).astype(jnp.int32)
    rel = (dst - c_e * CS).astype(jnp.int32)
    tab = jnp.zeros((128,), jnp.int32).at[0:nc].set(starts).at[64:64 + nc].set(cnt)
    return tab, rel, pos, e + nc * G


def kernel(ab_x, ab_edge_index, ab_batch, ba_x, ba_edge_index, ba_edge_attr,
           ba_batch, W1, b1, W2, b2, gamma, beta):
    n_ab = ab_x.shape[0]
    n_ba = ba_x.shape[0]
    e_ab = ab_edge_index.shape[1]
    e_ba = ba_edge_index.shape[1]
    ab_src, ab_dst = ab_edge_index[0], ab_edge_index[1]
    ba_src, ba_dst = ba_edge_index[0], ba_edge_index[1]
    ab_batch2 = ab_batch.reshape(-1, 1)
    ba_batch2 = ba_batch.reshape(-1, 1)
    n_ab_eff = ((n_ab + CS - 1) // CS) * CS
    if (n_ab_eff // CS) % 2:
        n_ab_eff += CS

    tab_ba, rel_ba, pos_ba, lpad_ba = _bucket(ba_dst, e_ba, n_ba)
    tab_ab, rel_ab, pos_ab, lpad_ab = _bucket(ab_dst, e_ab, n_ab_eff)

    srcg_ba, relg_ba, eidg_ba = _permute_kernel(e_ba, lpad_ba)(
        ba_src, rel_ba, pos_ba)
    srcg_ab, relg_ab, eidg_ab = _permute_kernel(e_ab, lpad_ab)(
        ab_src, rel_ab, pos_ab)

    zsmall = jnp.zeros((CS // 16, EMB), jnp.float32)
    s_attr = _segsum_ba_kernel(e_ba, lpad_ba, n_ba, True)(
        ba_edge_attr, eidg_ba, relg_ba, tab_ba, zsmall)

    _, cmat_ab = _counts(ab_batch2)
    _, cmat_ba = _counts(ba_batch2)
    scale_ab = _node_scale(ab_batch2, cmat_ab)
    scale_ba = _node_scale(ba_batch2, cmat_ba)

    ba_kern = _segsum_ba_kernel(n_ba, lpad_ba, n_ba, False)
    ab_kern1 = _segsum_ba_kernel(n_ab, lpad_ab, n_ab_eff, True)
    ab_kern2 = _segsum_ba_kernel(n_ba, lpad_ab, n_ab_eff, False)

    node_h, edge_h = ab_x, ba_x
    for l in range(LAYERS):
        last_act = (l != LAYERS - 1)
        part = ab_kern1(node_h, srcg_ab, relg_ab, tab_ab, zsmall)
        agg_ab = ab_kern2(edge_h, eidg_ab, relg_ab, tab_ab, part)
        agg_ba = ba_kern(edge_h, srcg_ba, relg_ba, tab_ba, s_attr)
        node_h = _mlp((agg_ab,), node_h, scale_ab, W1[l], b1[l],
                      W2[l], b2[l], gamma[l], beta[l], last_act)
        edge_h = _mlp((agg_ba,), edge_h, scale_ba, W1[l], b1[l], W2[l],
                      b2[l], gamma[l], beta[l], last_act)

    ab_repr = _pool(node_h, ab_batch2, cmat_ab)
    ba_repr = _pool(edge_h, ba_batch2, cmat_ba)
    return (ab_repr, ba_repr, node_h, edge_h)


# speedup vs baseline: 1.8843x; 1.7473x over previous
import functools

import jax
import jax.numpy as jnp
from jax import lax
from jax.experimental import pallas as pl
from jax.experimental.pallas import tpu as pltpu
from jax.experimental.pallas import tpu_sc as plsc

EMB = 128
HID = 256
NGRP = 256
LAYERS = 3

CS = 6400
G = 128


def _sc_mesh():
    return plsc.VectorSubcoreMesh(core_axis_name="c", subcore_axis_name="s")


def _permute2_kernel(e_ba, lpad_ba, e_ab, lpad_ab):
    assert e_ba % 16 == 0 and e_ab % 16 == 0
    assert lpad_ba % 128 == 0 and lpad_ab % 128 == 0 and lpad_ab <= lpad_ba
    D = 3

    scr = ([pltpu.VMEM((G,), jnp.int32) for _ in range(4 * D)]
           + [pltpu.SemaphoreType.DMA] * (2 * D)
           + [pltpu.VMEM_SHARED((lpad_ba,), jnp.int32) for _ in range(3)]
           + [pltpu.VMEM((32,), jnp.int32) for _ in range(4)]
           + [pltpu.VMEM((16,), jnp.int32) for _ in range(4)])

    @functools.partial(
        pl.kernel,
        out_type=tuple(jax.ShapeDtypeStruct((lp,), jnp.int32)
                       for lp in (lpad_ba, lpad_ba, lpad_ba,
                                  lpad_ab, lpad_ab, lpad_ab)),
        mesh=_sc_mesh(),
        scratch_types=scr,
    )
    def k(srcb_h, relb_h, posb_h, srca_h, rela_h, posa_h,
          sgb_h, rgb_h, egb_h, sga_h, rga_h, ega_h, *bufs):
        bufp = bufs[0:D]
        bufa = bufs[D:2 * D]
        bufb = bufs[2 * D:3 * D]
        bufe = bufs[3 * D:4 * D]
        sem_p = bufs[4 * D:5 * D]
        sem_o = bufs[5 * D:6 * D]
        sg_s, rg_s, eg_s = bufs[6 * D:6 * D + 3]
        tail32 = bufs[6 * D + 3:6 * D + 7]
        tail16 = bufs[6 * D + 7:6 * D + 11]
        cid = lax.axis_index("c")
        sid = lax.axis_index("s")

        def run_graph(src_h, rel_h, pos_h, e_cnt, tbufs):
            per_w = e_cnt // 16
            nfull = per_w // G
            tail = per_w - nfull * G
            base_w = sid * per_w

            def emit(i, r):
                @pl.when(i < nfull)
                def _():
                    @pl.when(i >= D)
                    def _():
                        pltpu.make_async_copy(bufa[r], sg_s.at[bufp[r]],
                                              sem_o[r]).wait()
                        pltpu.make_async_copy(bufb[r], rg_s.at[bufp[r]],
                                              sem_o[r]).wait()
                        pltpu.make_async_copy(bufe[r], eg_s.at[bufp[r]],
                                              sem_o[r]).wait()
                    base = base_w + i * G
                    pltpu.async_copy(pos_h.at[pl.ds(base, G)], bufp[r],
                                     sem_p[r])
                    pltpu.async_copy(src_h.at[pl.ds(base, G)], bufa[r],
                                     sem_p[r])
                    pltpu.async_copy(rel_h.at[pl.ds(base, G)], bufb[r],
                                     sem_p[r])
                    for q in range(G // 16):
                        bufe[r][pl.ds(q * 16, 16)] = (
                            base + q * 16
                            + lax.broadcasted_iota(jnp.int32, (16,), 0))
                r1 = (r - 1) % D
                @pl.when((i >= 1) & (i <= nfull))
                def _():
                    for _w in range(3):
                        pltpu.make_async_copy(pos_h.at[pl.ds(0, G)],
                                              bufp[r1], sem_p[r1]).wait()
                    pltpu.async_copy(bufa[r1], sg_s.at[bufp[r1]], sem_o[r1])
                    pltpu.async_copy(bufb[r1], rg_s.at[bufp[r1]], sem_o[r1])
                    pltpu.async_copy(bufe[r1], eg_s.at[bufp[r1]], sem_o[r1])

            def body(gi, carry):
                for r in range(D):
                    emit(gi * D + r, r)
                return carry

            lax.fori_loop(0, (nfull + 1 + (D - 1)) // D, body, 0)
            for sl in range(min(D, nfull)):
                pltpu.make_async_copy(bufa[sl], sg_s.at[bufp[sl]],
                                      sem_o[sl]).wait()
                pltpu.make_async_copy(bufb[sl], rg_s.at[bufp[sl]],
                                      sem_o[sl]).wait()
                pltpu.make_async_copy(bufe[sl], eg_s.at[bufp[sl]],
                                      sem_o[sl]).wait()

            if tail:
                assert tail % 16 == 0
                base = base_w + nfull * G
                tp, ta, tb, te = tbufs
                tw = tp.shape[0]
                assert tw == tail
                pltpu.sync_copy(pos_h.at[pl.ds(base, tail)], tp)
                pltpu.sync_copy(src_h.at[pl.ds(base, tail)], ta)
                pltpu.sync_copy(rel_h.at[pl.ds(base, tail)], tb)
                for q in range(tail // 16):
                    te[pl.ds(q * 16, 16)] = (
                        base + q * 16
                        + lax.broadcasted_iota(jnp.int32, (16,), 0))
                pltpu.sync_copy(ta, sg_s.at[tp])
                pltpu.sync_copy(tb, rg_s.at[tp])
                pltpu.sync_copy(te, eg_s.at[tp])

        @pl.when(cid == 0)
        def _():
            run_graph(srcb_h, relb_h, posb_h, e_ba, tail32)

        @pl.when(cid == 1)
        def _():
            run_graph(srca_h, rela_h, posa_h, e_ab, tail16)

        plsc.subcore_barrier()

        slb = lpad_ba // 16
        sla = lpad_ab // 16

        @pl.when(cid == 0)
        def _():
            pltpu.sync_copy(sg_s.at[pl.ds(sid * slb, slb)],
                            sgb_h.at[pl.ds(sid * slb, slb)])
            pltpu.sync_copy(rg_s.at[pl.ds(sid * slb, slb)],
                            rgb_h.at[pl.ds(sid * slb, slb)])
            pltpu.sync_copy(eg_s.at[pl.ds(sid * slb, slb)],
                            egb_h.at[pl.ds(sid * slb, slb)])

        @pl.when(cid == 1)
        def _():
            pltpu.sync_copy(sg_s.at[pl.ds(sid * sla, sla)],
                            sga_h.at[pl.ds(sid * sla, sla)])
            pltpu.sync_copy(rg_s.at[pl.ds(sid * sla, sla)],
                            rga_h.at[pl.ds(sid * sla, sla)])
            pltpu.sync_copy(eg_s.at[pl.ds(sid * sla, sla)],
                            ega_h.at[pl.ds(sid * sla, sla)])
        plsc.subcore_barrier()

    return k


_D = 4
_PIPE = True


def _segsum_ba_kernel(t_rows, l_pad, n_out, init_small):
    nc = n_out // CS
    assert n_out % CS == 0 and nc <= 64
    rows_t = CS // 16

    scr = ([pltpu.VMEM((128,), jnp.int32)]
           + [pltpu.VMEM((G,), jnp.int32) for _ in range(2 * _D)]
           + [pltpu.VMEM((G, EMB), jnp.float32) for _ in range(_D)]
           + [pltpu.VMEM_SHARED((CS + 1, EMB), jnp.float32)]
           + [pltpu.SemaphoreType.DMA] * (3 * _D))

    @functools.partial(
        pl.kernel,
        out_type=jax.ShapeDtypeStruct((n_out, EMB), jnp.float32),
        mesh=_sc_mesh(),
        scratch_types=scr,
    )
    def k(table_h, idxg_h, relg_h, tab_h, init_h, out_h, tab_v, *rest):
        idx_v = rest[0:_D]
        rel_v = rest[_D:2 * _D]
        rows_v = rest[2 * _D:3 * _D]
        acc = rest[3 * _D]
        sem_i = rest[3 * _D + 1:3 * _D + 1 + _D]
        sem_g = rest[3 * _D + 1 + _D:3 * _D + 1 + 2 * _D]
        sem_s = rest[3 * _D + 1 + 2 * _D:3 * _D + 1 + 3 * _D]
        cid = lax.axis_index("c")
        sid = lax.axis_index("s")
        pltpu.sync_copy(tab_h, tab_v)

        for cl in range((nc + 1) // 2):
            c = cl * 2 + cid
            g0 = (2 * cl) // 16
            l0 = (2 * cl) % 16
            sg = tab_v[pl.ds(g0 * 16, 16)]
            cg = tab_v[pl.ds(64 + g0 * 16, 16)]
            start = pl.multiple_of(jnp.where(cid == 1, sg[l0 + 1], sg[l0]), G)
            cnt = jnp.where(cid == 1, cg[l0 + 1], cg[l0])
            live = c < nc

            @pl.when(live)
            def _():
                if init_small:
                    pltpu.sync_copy(init_h,
                                    acc.at[pl.ds(sid * rows_t, rows_t)])
                else:
                    pltpu.sync_copy(
                        init_h.at[pl.ds(c * CS + sid * rows_t, rows_t)],
                        acc.at[pl.ds(sid * rows_t, rows_t)])
            plsc.subcore_barrier()

            nb = (cnt + (G - 1)) // G
            nb_t = (jnp.maximum(nb - sid, 0) + 15) // 16

            def emit(i, r):
                @pl.when(i < nb_t)
                def _():
                    @pl.when(i >= _D)
                    def _():
                        pltpu.make_async_copy(
                            rows_v[r], acc.at[rel_v[r]], sem_s[r]).wait()
                    j = sid + 16 * i
                    base = start + j * G
                    pltpu.async_copy(idxg_h.at[pl.ds(base, G)],
                                     idx_v[r], sem_i[r])
                    pltpu.async_copy(relg_h.at[pl.ds(base, G)],
                                     rel_v[r], sem_i[r])
                r1 = (r - 1) % _D
                @pl.when((i >= 1) & (i <= nb_t))
                def _():
                    m = i - 1
                    pltpu.make_async_copy(idxg_h.at[pl.ds(0, G)],
                                          idx_v[r1], sem_i[r1]).wait()
                    pltpu.make_async_copy(relg_h.at[pl.ds(0, G)],
                                          rel_v[r1], sem_i[r1]).wait()
                    j = sid + 16 * m
                    for q in range(G // 16):
                        off = (j * G + q * 16
                               + lax.broadcasted_iota(jnp.int32, (16,), 0))
                        valid = off < cnt
                        sl = pl.ds(q * 16, 16)
                        idx_v[r1][sl] = jnp.where(valid, idx_v[r1][sl], 0)
                        rel_v[r1][sl] = jnp.where(valid, rel_v[r1][sl], CS)
                    pltpu.async_copy(table_h.at[idx_v[r1]], rows_v[r1],
                                     sem_g[r1])
                r2 = (r - 2) % _D
                @pl.when((i >= 2) & (i <= nb_t + 1))
                def _():
                    pltpu.make_async_copy(table_h.at[idx_v[r2]],
                                          rows_v[r2], sem_g[r2]).wait()
                    pltpu.async_copy(rows_v[r2], acc.at[rel_v[r2]],
                                     sem_s[r2], add=True)

            def body(gi, carry):
                for r in range(_D):
                    emit(gi * _D + r, r)
                return carry

            if _PIPE:
                ngroups = (nb_t + 2 + (_D - 1)) // _D
                lax.fori_loop(0, ngroups, body, 0)
            else:
                def sbody(m, carry):
                    j = sid + 16 * m
                    base = start + j * G
                    pltpu.sync_copy(idxg_h.at[pl.ds(base, G)], idx_v[0])
                    pltpu.sync_copy(relg_h.at[pl.ds(base, G)], rel_v[0])
                    for q in range(G // 16):
                        off = (j * G + q * 16
                               + lax.broadcasted_iota(jnp.int32, (16,), 0))
                        valid = off < cnt
                        sl = pl.ds(q * 16, 16)
                        idx_v[0][sl] = jnp.where(valid, idx_v[0][sl], 0)
                        rel_v[0][sl] = jnp.where(valid, rel_v[0][sl], CS)
                    pltpu.async_copy(table_h.at[idx_v[0]], rows_v[0],
                                     sem_g[0]).wait()
                    pltpu.sync_copy(rows_v[0], acc.at[rel_v[0]], add=True)
                    return carry
                lax.fori_loop(0, nb_t, sbody, 0)
            if _PIPE:
                for s in range(_D):
                    @pl.when(nb_t > s)
                    def _():
                        pltpu.make_async_copy(
                            rows_v[s], acc.at[rel_v[s]], sem_s[s]).wait()
            plsc.subcore_barrier()

            @pl.when(live)
            def _():
                pltpu.sync_copy(
                    acc.at[pl.ds(sid * rows_t, rows_t)],
                    out_h.at[pl.ds(c * CS + sid * rows_t, rows_t)])
            plsc.subcore_barrier()

    return k


def _counts_body(batch_ref, row_ref, col_ref):
    i = pl.program_id(0)

    @pl.when(i == 0)
    def _():
        row_ref[...] = jnp.zeros_like(row_ref)
        col_ref[...] = jnp.zeros_like(col_ref)

    b = batch_ref[...]
    onehot = (b == lax.broadcasted_iota(jnp.int32, (1, NGRP), 1)).astype(jnp.float32)
    row_ref[0:1, :] += jnp.sum(onehot, axis=0, keepdims=True)
    col_ref[...] += lax.dot_general(
        onehot, jnp.ones((onehot.shape[0], 128), jnp.float32),
        (((0,), (0,)), ((), ())), preferred_element_type=jnp.float32)


def _counts(batch2d):
    n = batch2d.shape[0]
    B = 2048
    npad = pl.cdiv(n, B) * B
    bpad = jnp.full((npad, 1), NGRP, jnp.int32).at[:n].set(batch2d)
    return pl.pallas_call(
        _counts_body,
        grid=(npad // B,),
        in_specs=[pl.BlockSpec((B, 1), lambda i: (i, 0))],
        out_specs=[pl.BlockSpec((8, NGRP), lambda i: (0, 0)),
                   pl.BlockSpec((NGRP, 128), lambda i: (0, 0))],
        out_shape=[jax.ShapeDtypeStruct((8, NGRP), jnp.float32),
                   jax.ShapeDtypeStruct((NGRP, 128), jnp.float32)],
    )(bpad)


def _scale_body(batch_ref, cmat_ref, o_ref):
    b = batch_ref[...]
    onehot = (b == lax.broadcasted_iota(jnp.int32, (1, NGRP), 1)).astype(jnp.float32)
    rs = lax.rsqrt(jnp.maximum(cmat_ref[...][:, 0:1], 1.0))
    o_ref[...] = lax.dot_general(onehot, rs, (((1,), (0,)), ((), ())),
                                 preferred_element_type=jnp.float32)


def _node_scale(batch2d, cmat):
    n = batch2d.shape[0]
    B = 2048
    npad = pl.cdiv(n, B) * B
    bpad = jnp.full((npad, 1), NGRP, jnp.int32).at[:n].set(batch2d)
    out = pl.pallas_call(
        _scale_body,
        grid=(npad // B,),
        in_specs=[pl.BlockSpec((B, 1), lambda i: (i, 0)),
                  pl.BlockSpec((NGRP, 128), lambda i: (0, 0))],
        out_specs=pl.BlockSpec((B, 1), lambda i: (i, 0)),
        out_shape=jax.ShapeDtypeStruct((npad, 1), jnp.float32),
    )(bpad, cmat)
    return out[:n]


def _mlp_body(two_agg, agg_ref, agg2_ref, x_ref, scale_ref, W1_ref, b1_ref,
              W2_ref, b2_ref, g_ref, be_ref, o_ref, *, last_act):
    agg = agg_ref[...]
    if two_agg:
        agg = agg + agg2_ref[...]
    u = lax.dot_general(agg, W1_ref[...], (((1,), (0,)), ((), ())),
                        preferred_element_type=jnp.float32) + b1_ref[...]
    u = jnp.maximum(u, 0.0)
    h = lax.dot_general(u, W2_ref[...], (((1,), (0,)), ((), ())),
                        preferred_element_type=jnp.float32) + b2_ref[...]
    mu = jnp.mean(h, axis=1, keepdims=True)
    var = jnp.mean((h - mu) * (h - mu), axis=1, keepdims=True)
    h = (h - mu) * lax.rsqrt(var + 1e-5) * g_ref[...] + be_ref[...]
    h = h * scale_ref[...]
    if last_act:
        h = jnp.maximum(h, 0.0)
    o_ref[...] = h + x_ref[...]


def _mlp(aggs, x, scale, W1, b1, W2, b2, g, be, last_act):
    n = x.shape[0]
    B = 1024
    two = len(aggs) == 2
    body = functools.partial(_mlp_body, two, last_act=last_act)
    if not two:
        def body(agg_ref, *rest, _b=functools.partial(_mlp_body, False,
                                                      last_act=last_act)):
            _b(agg_ref, agg_ref, *rest)
    row = pl.BlockSpec((B, EMB), lambda i: (i, 0))
    return pl.pallas_call(
        body,
        grid=(pl.cdiv(n, B),),
        in_specs=([row] * (2 if two else 1)
                  + [row,
                     pl.BlockSpec((B, 1), lambda i: (i, 0)),
                     pl.BlockSpec((EMB, HID), lambda i: (0, 0)),
                     pl.BlockSpec((1, HID), lambda i: (0, 0)),
                     pl.BlockSpec((HID, EMB), lambda i: (0, 0)),
                     pl.BlockSpec((1, EMB), lambda i: (0, 0)),
                     pl.BlockSpec((1, EMB), lambda i: (0, 0)),
                     pl.BlockSpec((1, EMB), lambda i: (0, 0))]),
        out_specs=row,
        out_shape=jax.ShapeDtypeStruct((n, EMB), jnp.float32),
    )(*aggs, x, scale, W1, b1.reshape(1, HID), W2, b2.reshape(1, EMB),
      g.reshape(1, EMB), be.reshape(1, EMB))


def _pool_body(x_ref, batch_ref, cmat_ref, o_ref, *, nrows, nblocks, B):
    i = pl.program_id(0)

    @pl.when(i == 0)
    def _():
        o_ref[...] = jnp.zeros_like(o_ref)

    rowid = i * B + lax.broadcasted_iota(jnp.int32, (B, 1), 0)
    xm = jnp.where(rowid < nrows, x_ref[...], 0.0)
    b = batch_ref[...]
    onehot = (b == lax.broadcasted_iota(jnp.int32, (1, NGRP), 1)).astype(jnp.float32)
    o_ref[...] += lax.dot_general(onehot, xm, (((0,), (0,)), ((), ())),
                                  preferred_element_type=jnp.float32)

    @pl.when(i == nblocks - 1)
    def _():
        o_ref[...] = o_ref[...] / jnp.maximum(cmat_ref[...], 1.0)


def _pool(x, batch2d, cmat):
    n = x.shape[0]
    B = 2048
    npad = pl.cdiv(n, B) * B
    nblocks = npad // B
    bpad = jnp.full((npad, 1), NGRP, jnp.int32).at[:n].set(batch2d)
    body = functools.partial(_pool_body, nrows=n, nblocks=nblocks, B=B)
    return pl.pallas_call(
        body,
        grid=(nblocks,),
        in_specs=[pl.BlockSpec((B, EMB), lambda i: (i, 0)),
                  pl.BlockSpec((B, 1), lambda i: (i, 0)),
                  pl.BlockSpec((NGRP, 128), lambda i: (0, 0))],
        out_specs=pl.BlockSpec((NGRP, EMB), lambda i: (0, 0)),
        out_shape=jax.ShapeDtypeStruct((NGRP, EMB), jnp.float32),
    )(x, bpad, cmat)


def _bucket(dst, e, n_eff):
    nc = n_eff // CS
    assert nc <= 64
    c_e = dst // CS
    oh = (c_e[:, None] == jnp.arange(nc, dtype=jnp.int32)[None, :]).astype(jnp.int32)
    rank = jnp.sum((jnp.cumsum(oh, axis=0) - oh) * oh, axis=1)
    cnt = jnp.sum(oh, axis=0).astype(jnp.int32)
    cnt_pad = ((cnt + (G - 1)) // G) * G
    starts = jnp.concatenate([jnp.zeros((1,), jnp.int32),
                              jnp.cumsum(cnt_pad)[:-1].astype(jnp.int32)])
    pos = (jnp.sum(starts[None, :] * oh, axis=1) + rank).astype(jnp.int32)
    rel = (dst - c_e * CS).astype(jnp.int32)
    tab = jnp.zeros((128,), jnp.int32).at[0:nc].set(starts).at[64:64 + nc].set(cnt)
    return tab, rel, pos, ((e + nc * G + 2047) // 2048) * 2048


def kernel(ab_x, ab_edge_index, ab_batch, ba_x, ba_edge_index, ba_edge_attr,
           ba_batch, W1, b1, W2, b2, gamma, beta):
    n_ab = ab_x.shape[0]
    n_ba = ba_x.shape[0]
    e_ab = ab_edge_index.shape[1]
    e_ba = ba_edge_index.shape[1]
    ab_src, ab_dst = ab_edge_index[0], ab_edge_index[1]
    ba_src, ba_dst = ba_edge_index[0], ba_edge_index[1]
    ab_batch2 = ab_batch.reshape(-1, 1)
    ba_batch2 = ba_batch.reshape(-1, 1)
    n_ab_eff = ((n_ab + CS - 1) // CS) * CS
    if (n_ab_eff // CS) % 2:
        n_ab_eff += CS

    tab_ba, rel_ba, pos_ba, lpad_ba = _bucket(ba_dst, e_ba, n_ba)
    tab_ab, rel_ab, pos_ab, lpad_ab = _bucket(ab_dst, e_ab, n_ab_eff)

    (srcg_ba, relg_ba, eidg_ba, srcg_ab, relg_ab, eidg_ab) = \
        _permute2_kernel(e_ba, lpad_ba, e_ab, lpad_ab)(
            ba_src, rel_ba, pos_ba, ab_src, rel_ab, pos_ab)

    zsmall = jnp.zeros((CS // 16, EMB), jnp.float32)
    s_attr = _segsum_ba_kernel(e_ba, lpad_ba, n_ba, True)(
        ba_edge_attr, eidg_ba, relg_ba, tab_ba, zsmall)

    _, cmat_ab = _counts(ab_batch2)
    _, cmat_ba = _counts(ba_batch2)
    scale_ab = _node_scale(ab_batch2, cmat_ab)
    scale_ba = _node_scale(ba_batch2, cmat_ba)

    ba_kern = _segsum_ba_kernel(n_ba, lpad_ba, n_ba, False)
    ab_kern1 = _segsum_ba_kernel(n_ab, lpad_ab, n_ab_eff, True)
    ab_kern2 = _segsum_ba_kernel(n_ba, lpad_ab, n_ab_eff, False)

    node_h, edge_h = ab_x, ba_x
    for l in range(LAYERS):
        last_act = (l != LAYERS - 1)
        part = ab_kern1(node_h, srcg_ab, relg_ab, tab_ab, zsmall)
        agg_ab = ab_kern2(edge_h, eidg_ab, relg_ab, tab_ab, part)
        agg_ba = ba_kern(edge_h, srcg_ba, relg_ba, tab_ba, s_attr)
        node_h = _mlp((agg_ab,), node_h, scale_ab, W1[l], b1[l],
                      W2[l], b2[l], gamma[l], beta[l], last_act)
        edge_h = _mlp((agg_ba,), edge_h, scale_ba, W1[l], b1[l], W2[l],
                      b2[l], gamma[l], beta[l], last_act)

    ab_repr = _pool(node_h, ab_batch2, cmat_ab)
    ba_repr = _pool(edge_h, ba_batch2, cmat_ba)
    return (ab_repr, ba_repr, node_h, edge_h)
